# baseline (speedup 1.0000x reference)
import functools

import jax
import jax.numpy as jnp
from jax.experimental import pallas as pl
from jax.experimental.pallas import tpu as pltpu

EPS = 1e-5
LANES = 128
_NB2 = 16384


def _lane_fold(h, nl):
    acc = h[:, 0:LANES]
    for j in range(1, nl // LANES):
        acc = acc + h[:, j * LANES:(j + 1) * LANES]
    return acc


def _h2_pair(zq_ref, w1d_ref, b1d_ref, w2d_ref, b2d_ref):
    h1 = jnp.dot(w1d_ref[...], zq_ref[...], preferred_element_type=jnp.float32)
    h1 = jnp.maximum(h1 + b1d_ref[...], 0.0)
    h2 = jnp.dot(w2d_ref[...], h1, preferred_element_type=jnp.float32)
    return jnp.maximum(h2 + b2d_ref[...], 0.0)


def _accum_stats(s_ref, h, nl):
    @pl.when(pl.program_id(0) == 0)
    def _():
        s_ref[...] = jnp.zeros_like(s_ref)

    d = h.shape[0] // 2
    hh = h * h
    s_ref[0] += _lane_fold(h[:d] + h[d:], nl)
    s_ref[1] += _lane_fold(hh[:d] + hh[d:], nl)


def _stats1_kernel(zq_ref, w1d_ref, b1d_ref, s_ref, *, nb2):
    h1 = jnp.dot(w1d_ref[...], zq_ref[...], preferred_element_type=jnp.float32)
    h1 = jnp.maximum(h1 + b1d_ref[...], 0.0)
    _accum_stats(s_ref, h1, nb2)


def _stats2_kernel(zq_ref, w1d_ref, b1d_ref, w2d_ref, b2d_ref, s_ref, *, nb2):
    _accum_stats(
        s_ref, _h2_pair(zq_ref, w1d_ref, b1d_ref, w2d_ref, b2d_ref), nb2)


def _out_kernel(zq_ref, w1d_ref, b1d_ref, w2d_ref, b2d_ref, w3d_ref, b3d_ref,
                o_ref, *, nb2):
    h2 = _h2_pair(zq_ref, w1d_ref, b1d_ref, w2d_ref, b2d_ref)
    w3d = w3d_ref[...]
    b3d = b3d_ref[...]
    for j in range(nb2 // 256):
        blk = jax.lax.dot_general(
            h2[:, j * 256:(j + 1) * 256], w3d,
            dimension_numbers=(((0,), (0,)), ((), ())),
            preferred_element_type=jnp.float32)
        blk = jax.nn.sigmoid(blk + b3d)
        o_ref[j * 512:j * 512 + 256, :] = blk[:, 0:LANES]
        o_ref[j * 512 + 256:j * 512 + 512, :] = blk[:, LANES:2 * LANES]


def _bn_fold(stats, inv_b, g, be, w, b):
    st = jnp.sum(stats, axis=2)
    m = st[0] * inv_b
    var = jnp.maximum(st[1] * inv_b - m * m, 0.0)
    scale = g * jax.lax.rsqrt(var + EPS)
    shift = be - m * scale
    return scale[:, None] * w, shift @ w + b


def _double(w, b):
    r, c = w.shape
    wd = jnp.zeros((2 * r, 2 * c), w.dtype)
    wd = wd.at[:r, :c].set(w).at[r:, c:].set(w)
    bd = jnp.concatenate([b, b])
    return wd, bd[:, None]


def kernel(z, slab):
    r2, r3 = 16, 144
    d2, d1, d0 = 32, 64, 128
    B = z.shape[0]
    nb2 = _NB2
    while (B // 2) % nb2:
        nb2 //= 2
    T = B // (2 * nb2)
    inv_b = 1.0 / B

    w1t = jnp.transpose(jax.lax.slice(slab, (0, 0), (2, d2)))
    b1 = slab[2, :d2]
    g1, be1 = slab[3, :d2], slab[4, :d2]
    b2 = slab[5, :d1]
    g2, be2 = slab[6, :d1], slab[7, :d1]
    b3 = slab[8, :]
    w2 = jax.lax.slice(slab, (r2, 0), (r2 + d2, d1))
    w3 = jax.lax.slice(slab, (r3, 0), (r3 + d1, d0))

    zq = (jnp.transpose(z).reshape(2, B // 512, 2, 256)
          .transpose(2, 0, 1, 3).reshape(4, B // 2))
    w1d, b1d = _double(w1t, b1)

    arb = pltpu.CompilerParams(dimension_semantics=("arbitrary",))
    zq_spec = pl.BlockSpec((4, nb2), lambda t: (0, t))
    small = lambda a: pl.BlockSpec(a.shape, lambda t: (0,) * a.ndim)

    s1 = pl.pallas_call(
        functools.partial(_stats1_kernel, nb2=nb2),
        grid=(T,),
        out_shape=jax.ShapeDtypeStruct((2, d2, LANES), jnp.float32),
        in_specs=[zq_spec, small(w1d), small(b1d)],
        out_specs=pl.BlockSpec((2, d2, LANES), lambda t: (0, 0, 0)),
        compiler_params=arb,
    )(zq, w1d, b1d)

    w2f, b2f = _bn_fold(s1, inv_b, g1, be1, w2, b2)
    w2d, b2d = _double(jnp.transpose(w2f), b2f)

    s2 = pl.pallas_call(
        functools.partial(_stats2_kernel, nb2=nb2),
        grid=(T,),
        out_shape=jax.ShapeDtypeStruct((2, d1, LANES), jnp.float32),
        in_specs=[zq_spec, small(w1d), small(b1d), small(w2d), small(b2d)],
        out_specs=pl.BlockSpec((2, d1, LANES), lambda t: (0, 0, 0)),
        compiler_params=arb,
    )(zq, w1d, b1d, w2d, b2d)

    w3f, b3f = _bn_fold(s2, inv_b, g2, be2, w3, b3)
    w3d, b3d = _double(w3f, b3f)
    b3d = jnp.transpose(b3d)

    out = pl.pallas_call(
        functools.partial(_out_kernel, nb2=nb2),
        grid=(T,),
        out_shape=jax.ShapeDtypeStruct((B, LANES), jnp.float32),
        in_specs=[zq_spec, small(w1d), small(b1d), small(w2d), small(b2d),
                  small(w3d), small(b3d)],
        out_specs=pl.BlockSpec((2 * nb2, LANES), lambda t: (t, 0)),
        compiler_params=arb,
    )(zq, w1d, b1d, w2d, b2d, w3d, b3d)
    return out


# speedup vs baseline: 1.0969x; 1.0969x over previous
import functools

import jax
import jax.numpy as jnp
from jax.experimental import pallas as pl
from jax.experimental.pallas import tpu as pltpu

EPS = 1e-5
LANES = 128
_NB = 32768


def _lane_fold(h, nb):
    acc = h[:, 0:LANES]
    for j in range(1, nb // LANES):
        acc = acc + h[:, j * LANES:(j + 1) * LANES]
    return acc


def _h1t(zt_ref, w1p_ref):
    z0 = zt_ref[0:1, :]
    z1 = zt_ref[1:2, :]
    h = w1p_ref[:, 0:1] * z0 + w1p_ref[:, 1:2] * z1 + w1p_ref[:, 2:3]
    return jnp.maximum(h, 0.0)


def _h2t(h1t, w2t_ref, b2c_ref):
    h = jnp.dot(w2t_ref[...], h1t, preferred_element_type=jnp.float32)
    return jnp.maximum(h + b2c_ref[...], 0.0)


def _accum_stats(s_ref, h, nb):
    @pl.when(pl.program_id(0) == 0)
    def _():
        s_ref[...] = jnp.zeros_like(s_ref)

    s_ref[0] += _lane_fold(h, nb)
    s_ref[1] += _lane_fold(h * h, nb)


def _stats1_kernel(zt_ref, w1t_ref, b1c_ref, s_ref, *, nb):
    h = jnp.dot(w1t_ref[...], zt_ref[...], preferred_element_type=jnp.float32)
    h = jnp.maximum(h + b1c_ref[...], 0.0)
    _accum_stats(s_ref, h, nb)


def _stats2_kernel(zt_ref, w1p_ref, w2t_ref, b2c_ref, s_ref, h2_ref, *, nb):
    h2 = _h2t(_h1t(zt_ref, w1p_ref), w2t_ref, b2c_ref)
    _accum_stats(s_ref, h2, nb)
    h2_ref[...] = h2.astype(jnp.bfloat16)


def _out_kernel(h2_ref, w3d_ref, b3d_ref, o_ref, *, nb):
    w3d = w3d_ref[...]
    b3d = b3d_ref[...]
    for j in range(nb // 512):
        lo, hi = j * 512, j * 512 + 256
        pair = jnp.concatenate(
            [h2_ref[:, lo:hi], h2_ref[:, hi:hi + 256]], axis=0)
        blk = jax.lax.dot_general(
            pair, w3d, dimension_numbers=(((0,), (0,)), ((), ())),
            preferred_element_type=jnp.float32)
        blk = jax.nn.sigmoid(blk + b3d)
        o_ref[lo:hi, :] = blk[:, 0:LANES]
        o_ref[hi:hi + 256, :] = blk[:, LANES:2 * LANES]


def _bn_fold(stats, inv_b, g, be, w, b):
    st = jnp.sum(stats, axis=2)
    m = st[0] * inv_b
    var = jnp.maximum(st[1] * inv_b - m * m, 0.0)
    scale = g * jax.lax.rsqrt(var + EPS)
    shift = be - m * scale
    return scale[:, None] * w, shift @ w + b


def kernel(z, slab):
    r2, r3 = 16, 144
    d2, d1, d0 = 32, 64, 128
    B = z.shape[0]
    nb = _NB
    while B % nb:
        nb //= 2
    T = B // nb
    inv_b = 1.0 / B

    w1p = jnp.transpose(jax.lax.slice(slab, (0, 0), (3, d2)))
    w1t = jax.lax.slice(w1p, (0, 0), (d2, 2))
    b1c = jax.lax.slice(w1p, (0, 2), (d2, 3))
    g1, be1 = slab[3, :d2], slab[4, :d2]
    b2 = slab[5, :d1]
    g2, be2 = slab[6, :d1], slab[7, :d1]
    b3 = slab[8, :]
    w2 = jax.lax.slice(slab, (r2, 0), (r2 + d2, d1))
    w3 = jax.lax.slice(slab, (r3, 0), (r3 + d1, d0))

    zt = jnp.transpose(z)

    arb = pltpu.CompilerParams(dimension_semantics=("arbitrary",))
    zt_spec = pl.BlockSpec((2, nb), lambda t: (0, t))
    h2_spec = pl.BlockSpec((d1, nb), lambda t: (0, t))
    small = lambda a: pl.BlockSpec(a.shape, lambda t: (0,) * a.ndim)

    s1 = pl.pallas_call(
        functools.partial(_stats1_kernel, nb=nb),
        grid=(T,),
        out_shape=jax.ShapeDtypeStruct((2, d2, LANES), jnp.float32),
        in_specs=[zt_spec, small(w1t), small(b1c)],
        out_specs=pl.BlockSpec((2, d2, LANES), lambda t: (0, 0, 0)),
        compiler_params=arb,
    )(zt, w1t, b1c)

    w2f, b2f = _bn_fold(s1, inv_b, g1, be1, w2, b2)
    w2t = jnp.transpose(w2f)
    b2c = b2f[:, None]

    s2, h2t = pl.pallas_call(
        functools.partial(_stats2_kernel, nb=nb),
        grid=(T,),
        out_shape=(jax.ShapeDtypeStruct((2, d1, LANES), jnp.float32),
                   jax.ShapeDtypeStruct((d1, B), jnp.bfloat16)),
        in_specs=[zt_spec, small(w1p), small(w2t), small(b2c)],
        out_specs=(pl.BlockSpec((2, d1, LANES), lambda t: (0, 0, 0)), h2_spec),
        compiler_params=arb,
    )(zt, w1p, w2t, b2c)

    w3f, b3f = _bn_fold(s2, inv_b, g2, be2, w3, b3)
    w3d = jnp.zeros((2 * d1, 2 * d0), jnp.float32)
    w3d = w3d.at[:d1, :d0].set(w3f).at[d1:, d0:].set(w3f)
    w3d = w3d.astype(jnp.bfloat16)
    b3d = jnp.concatenate([b3f, b3f])[None, :]

    out = pl.pallas_call(
        functools.partial(_out_kernel, nb=nb),
        grid=(T,),
        out_shape=jax.ShapeDtypeStruct((B, LANES), jnp.float32),
        in_specs=[h2_spec, small(w3d), small(b3d)],
        out_specs=pl.BlockSpec((nb, LANES), lambda t: (t, 0)),
        compiler_params=arb,
    )(h2t, w3d, b3d)
    return out


# speedup vs baseline: 1.1142x; 1.0158x over previous
import functools

import jax
import jax.numpy as jnp
from jax.experimental import pallas as pl
from jax.experimental.pallas import tpu as pltpu

EPS = 1e-5
LANES = 128
_NB = 32768


def _lane_fold(h, nb):
    acc = h[:, 0:LANES]
    for j in range(1, nb // LANES):
        acc = acc + h[:, j * LANES:(j + 1) * LANES]
    return acc


def _h1t(zt_ref, w1p_ref):
    z0 = zt_ref[0:1, :]
    z1 = zt_ref[1:2, :]
    h = w1p_ref[:, 0:1] * z0 + w1p_ref[:, 1:2] * z1 + w1p_ref[:, 2:3]
    return jnp.maximum(h, 0.0)


def _h2t(h1t, w2t_ref, b2c_ref):
    h = jnp.dot(w2t_ref[...], h1t, preferred_element_type=jnp.float32)
    return jnp.maximum(h + b2c_ref[...], 0.0)


def _accum_stats(s_ref, h, nb):
    @pl.when(pl.program_id(0) == 0)
    def _():
        s_ref[...] = jnp.zeros_like(s_ref)

    s_ref[0] += _lane_fold(h, nb)
    s_ref[1] += _lane_fold(h * h, nb)


def _stats1_kernel(zt_ref, w1t_ref, b1c_ref, s_ref, *, nb):
    h = jnp.dot(w1t_ref[...], zt_ref[...], preferred_element_type=jnp.float32)
    h = jnp.maximum(h + b1c_ref[...], 0.0)
    _accum_stats(s_ref, h, nb)


def _stats2_kernel(zt_ref, w1p_ref, w2t_ref, b2c_ref, s_ref, h2_ref, *, nb):
    h2 = _h2t(_h1t(zt_ref, w1p_ref), w2t_ref, b2c_ref)
    _accum_stats(s_ref, h2, nb)
    h2_ref[...] = h2.astype(jnp.bfloat16)


def _out_kernel(h2_ref, w3d_ref, b3d_ref, o_ref, *, nb):
    w3d = w3d_ref[...]
    b3d = b3d_ref[...]
    for j in range(nb // 512):
        lo, hi = j * 512, j * 512 + 256
        pair = jnp.concatenate(
            [h2_ref[:, lo:hi], h2_ref[:, hi:hi + 256]], axis=0)
        blk = jax.lax.dot_general(
            pair, w3d, dimension_numbers=(((0,), (0,)), ((), ())),
            preferred_element_type=jnp.float32)
        blk = jax.nn.sigmoid(blk + b3d)
        o_ref[lo:hi, :] = blk[:, 0:LANES]
        o_ref[hi:hi + 256, :] = blk[:, LANES:2 * LANES]


def _bn_fold(stats, inv_b, g, be, w, b):
    st = jnp.sum(stats, axis=2)
    m = st[0] * inv_b
    var = jnp.maximum(st[1] * inv_b - m * m, 0.0)
    scale = g * jax.lax.rsqrt(var + EPS)
    shift = be - m * scale
    return scale[:, None] * w, shift @ w + b


def kernel(z, slab):
    r2, r3 = 16, 144
    d2, d1, d0 = 32, 64, 128
    B = z.shape[0]
    nb = _NB
    while B % nb:
        nb //= 2
    T = B // nb
    nbs = 2 * nb if B % (2 * nb) == 0 else nb
    Ts = B // nbs
    inv_b = 1.0 / B

    w1p = jnp.transpose(jax.lax.slice(slab, (0, 0), (3, d2)))
    w1t = jax.lax.slice(w1p, (0, 0), (d2, 2))
    b1c = jax.lax.slice(w1p, (0, 2), (d2, 3))
    g1, be1 = slab[3, :d2], slab[4, :d2]
    b2 = slab[5, :d1]
    g2, be2 = slab[6, :d1], slab[7, :d1]
    b3 = slab[8, :]
    w2 = jax.lax.slice(slab, (r2, 0), (r2 + d2, d1))
    w3 = jax.lax.slice(slab, (r3, 0), (r3 + d1, d0))

    zt = jnp.transpose(z)

    arb = pltpu.CompilerParams(dimension_semantics=("arbitrary",))
    zts_spec = pl.BlockSpec((2, nbs), lambda t: (0, t))
    h2_spec = pl.BlockSpec((d1, nb), lambda t: (0, t))
    h2s_spec = pl.BlockSpec((d1, nbs), lambda t: (0, t))
    small = lambda a: pl.BlockSpec(a.shape, lambda t: (0,) * a.ndim)

    s1 = pl.pallas_call(
        functools.partial(_stats1_kernel, nb=nbs),
        grid=(Ts,),
        out_shape=jax.ShapeDtypeStruct((2, d2, LANES), jnp.float32),
        in_specs=[zts_spec, small(w1t), small(b1c)],
        out_specs=pl.BlockSpec((2, d2, LANES), lambda t: (0, 0, 0)),
        compiler_params=arb,
    )(zt, w1t, b1c)

    w2f, b2f = _bn_fold(s1, inv_b, g1, be1, w2, b2)
    w2t = jnp.transpose(w2f)
    b2c = b2f[:, None]

    s2, h2t = pl.pallas_call(
        functools.partial(_stats2_kernel, nb=nbs),
        grid=(Ts,),
        out_shape=(jax.ShapeDtypeStruct((2, d1, LANES), jnp.float32),
                   jax.ShapeDtypeStruct((d1, B), jnp.bfloat16)),
        in_specs=[zts_spec, small(w1p), small(w2t), small(b2c)],
        out_specs=(pl.BlockSpec((2, d1, LANES), lambda t: (0, 0, 0)),
                   h2s_spec),
        compiler_params=arb,
    )(zt, w1p, w2t, b2c)

    w3f, b3f = _bn_fold(s2, inv_b, g2, be2, w3, b3)
    w3d = jnp.zeros((2 * d1, 2 * d0), jnp.float32)
    w3d = w3d.at[:d1, :d0].set(w3f).at[d1:, d0:].set(w3f)
    w3d = w3d.astype(jnp.bfloat16)
    b3d = jnp.concatenate([b3f, b3f])[None, :]

    out = pl.pallas_call(
        functools.partial(_out_kernel, nb=nb),
        grid=(T,),
        out_shape=jax.ShapeDtypeStruct((B, LANES), jnp.float32),
        in_specs=[h2_spec, small(w3d), small(b3d)],
        out_specs=pl.BlockSpec((nb, LANES), lambda t: (t, 0)),
        compiler_params=arb,
    )(h2t, w3d, b3d)
    return out
